# baseline (speedup 1.0000x reference)
import functools

import jax
import jax.numpy as jnp
from jax import lax
from jax.experimental import pallas as pl
from jax.experimental.pallas import tpu as pltpu
from jax.experimental.pallas import tpu_sc as plsc

N = 10000
E = 320000
D = 128
DW = 16
NC, NS = 2, 16
NW = NC * NS
B = 128
EP = 327680
EPW = EP // NW
NB = EPW // B
RPS = 640
NP = NS * RPS

_mesh = plsc.VectorSubcoreMesh(
    core_axis_name="c", subcore_axis_name="s", num_cores=NC, num_subcores=NS)


@functools.partial(
    pl.kernel,
    out_type=jax.ShapeDtypeStruct((NC, NP), jnp.float32),
    mesh=_mesh,
    scratch_types=[
        pltpu.VMEM_SHARED((NP,), jnp.float32),
        pltpu.VMEM((2, B), jnp.int32),
        pltpu.VMEM((B,), jnp.float32),
        pltpu.SemaphoreType.DMA,
    ],
)
def _deg_kernel(dst1d, ones_hbm, zeros_hbm, out, deg_sh, didx2, ones_v, isem):
    c = lax.axis_index("c")
    s = lax.axis_index("s")
    w = c * NS + s
    pltpu.sync_copy(ones_hbm, ones_v)
    pltpu.sync_copy(zeros_hbm.at[pl.ds(s * RPS, RPS)],
                    deg_sh.at[pl.ds(s * RPS, RPS)])
    pltpu.sync_copy(dst1d.at[pl.ds(w * EPW, B)], didx2.at[0])
    plsc.subcore_barrier()

    def body(j, carry):
        b = lax.rem(j, 2)
        nb = 1 - b
        jn = jnp.minimum(j + 1, NB - 1)
        iad = pltpu.async_copy(
            dst1d.at[pl.ds(w * EPW + jn * B, B)], didx2.at[nb], isem)
        pltpu.sync_copy(ones_v, deg_sh.at[didx2.at[b]], add=True)
        iad.wait()
        return carry

    lax.fori_loop(0, NB, body, None)
    plsc.subcore_barrier()
    pltpu.sync_copy(deg_sh.at[pl.ds(s * RPS, RPS)],
                    out.at[c, pl.ds(s * RPS, RPS)])


@functools.partial(
    pl.kernel,
    out_type=jax.ShapeDtypeStruct((NC, NP, D), jnp.float32),
    mesh=_mesh,
    scratch_types=[
        pltpu.VMEM_SHARED((NP, D), jnp.float32),
        pltpu.VMEM((2, B), jnp.int32),
        pltpu.VMEM((2, B), jnp.int32),
        pltpu.VMEM((2, B, D), jnp.float32),
        pltpu.SemaphoreType.DMA,
    ],
)
def _agg_kernel(src1d, dst1d, g_hbm, zeros_hbm, out, acc_sh, sidx2, didx2,
                rows2, gsem):
    c = lax.axis_index("c")
    s = lax.axis_index("s")
    w = c * NS + s
    pltpu.sync_copy(zeros_hbm.at[pl.ds(s * RPS, RPS)],
                    acc_sh.at[pl.ds(s * RPS, RPS)])
    pltpu.sync_copy(src1d.at[pl.ds(w * EPW, B)], sidx2.at[0])
    pltpu.sync_copy(dst1d.at[pl.ds(w * EPW, B)], didx2.at[0])
    plsc.subcore_barrier()

    def body(j, carry):
        b = lax.rem(j, 2)
        nb = 1 - b
        gd = pltpu.async_copy(g_hbm.at[sidx2.at[b]], rows2.at[b], gsem)

        @pl.when(j >= 1)
        def _():
            pltpu.sync_copy(rows2.at[nb], acc_sh.at[didx2.at[nb]], add=True)

        jn = jnp.minimum(j + 1, NB - 1)
        e0 = w * EPW + jn * B
        pltpu.sync_copy(src1d.at[pl.ds(e0, B)], sidx2.at[nb])
        pltpu.sync_copy(dst1d.at[pl.ds(e0, B)], didx2.at[nb])
        gd.wait()
        return carry

    lax.fori_loop(0, NB, body, None)
    pltpu.sync_copy(rows2.at[(NB - 1) % 2], acc_sh.at[didx2.at[(NB - 1) % 2]],
                    add=True)
    plsc.subcore_barrier()
    pltpu.sync_copy(acc_sh.at[pl.ds(s * RPS, RPS)],
                    out.at[c, pl.ds(s * RPS, RPS)])


_RB = 1000


def _scale_body(x_ref, dga_ref, dgb_ref, g_ref):
    deg = dga_ref[...] + dgb_ref[...] + 1.0
    g_ref[...] = x_ref[...] * lax.rsqrt(deg)


def _tc_scale(x, dga, dgb):
    return pl.pallas_call(
        _scale_body,
        grid=(N // _RB,),
        in_specs=[
            pl.BlockSpec((_RB, D), lambda i: (i, 0)),
            pl.BlockSpec((_RB, 1), lambda i: (i, 0)),
            pl.BlockSpec((_RB, 1), lambda i: (i, 0)),
        ],
        out_specs=pl.BlockSpec((_RB, D), lambda i: (i, 0)),
        out_shape=jax.ShapeDtypeStruct((N, D), jnp.float32),
    )(x, dga, dgb)


def _final_body(acca_ref, accb_ref, g_ref, dga_ref, dgb_ref, w_ref, b_ref,
                o_ref):
    deg = dga_ref[...] + dgb_ref[...] + 1.0
    t = (acca_ref[...] + accb_ref[...] + g_ref[...]) * lax.rsqrt(deg)
    h = jnp.dot(t, w_ref[...], preferred_element_type=jnp.float32)
    h = h + b_ref[...]
    val = h[:, :D]
    gate = h[:, D:]
    o_ref[...] = val * (0.5 * gate * (1.0 + lax.erf(gate * 0.7071067811865476)))


def _tc_final(acca, accb, g, dga, dgb, W, b2):
    return pl.pallas_call(
        _final_body,
        grid=(N // _RB,),
        in_specs=[
            pl.BlockSpec((_RB, D), lambda i: (i, 0)),
            pl.BlockSpec((_RB, D), lambda i: (i, 0)),
            pl.BlockSpec((_RB, D), lambda i: (i, 0)),
            pl.BlockSpec((_RB, 1), lambda i: (i, 0)),
            pl.BlockSpec((_RB, 1), lambda i: (i, 0)),
            pl.BlockSpec((D, 2 * D), lambda i: (0, 0)),
            pl.BlockSpec((1, 2 * D), lambda i: (0, 0)),
        ],
        out_specs=pl.BlockSpec((_RB, D), lambda i: (i, 0)),
        out_shape=jax.ShapeDtypeStruct((N, D), jnp.float32),
    )(acca, accb, g, dga, dgb, W, b2)


def kernel(x, edge_index, W, b):
    npad = EP - E
    pad_src = jnp.zeros((npad,), jnp.int32)
    pad_dst = N + (jnp.arange(npad, dtype=jnp.int32) % (NP - N))
    src1d = jnp.concatenate([edge_index[0].astype(jnp.int32), pad_src])
    dst1d = jnp.concatenate([edge_index[1].astype(jnp.int32), pad_dst])
    ones1 = jnp.ones((B,), jnp.float32)
    zdeg = jnp.zeros((NP,), jnp.float32)
    zacc = jnp.zeros((NP, D), jnp.float32)

    degp = _deg_kernel(dst1d, ones1, zdeg)
    dga = degp[0].reshape(NP, 1)
    dgb = degp[1].reshape(NP, 1)
    g = _tc_scale(x, dga, dgb)
    accp = _agg_kernel(src1d, dst1d, g, zacc)
    return _tc_final(accp[0], accp[1], g, dga, dgb, W,
                     b.reshape(1, 2 * D))
